# Initial kernel scaffold; baseline (speedup 1.0000x reference)
import jax
import jax.numpy as jnp
from jax.experimental import pallas as pl


def kernel(x, edge_index, edge_attr, W1, b1, W2, b2, Wu, bu, gamma, beta):
    raise NotImplementedError("write your pallas kernel here")


# speedup vs baseline: 3.5024x; 3.5024x over previous
import functools

import jax
import jax.numpy as jnp
from jax import lax
from jax.experimental import pallas as pl
from jax.experimental.pallas import tpu as pltpu
from jax.experimental.pallas import tpu_sc as plsc

D = 128
NC = 2
NS = 16
NW = NC * NS
CHUNK = 128


def _mlp_body(x_ref, w1_ref, b1_ref, w2_ref, b2_ref, o_ref):
    h = jnp.dot(x_ref[...], w1_ref[...], preferred_element_type=jnp.float32)
    h = jnp.maximum(h + b1_ref[...], 0.0)
    o_ref[...] = jnp.dot(h, w2_ref[...],
                         preferred_element_type=jnp.float32) + b2_ref[...]


def _mlp(x, W1, b1, W2, b2):
    n = x.shape[0]
    blk = 1000
    return pl.pallas_call(
        _mlp_body,
        grid=(n // blk,),
        in_specs=[
            pl.BlockSpec((blk, D), lambda i: (i, 0)),
            pl.BlockSpec((D, D), lambda i: (0, 0)),
            pl.BlockSpec((1, D), lambda i: (0, 0)),
            pl.BlockSpec((D, D), lambda i: (0, 0)),
            pl.BlockSpec((1, D), lambda i: (0, 0)),
        ],
        out_specs=pl.BlockSpec((blk, D), lambda i: (i, 0)),
        out_shape=jax.ShapeDtypeStruct((n, D), jnp.float32),
    )(x, W1, b1.reshape(1, D), W2, b2.reshape(1, D))


def _sc_scatter(t, src, dst, attr, zeros_block, n_pad):
    rows_per_tile = n_pad // NS
    ep = src.shape[0]
    chunks = ep // (NW * CHUNK)
    mesh = plsc.VectorSubcoreMesh(core_axis_name="c", subcore_axis_name="s")

    @functools.partial(
        pl.kernel,
        mesh=mesh,
        out_type=jax.ShapeDtypeStruct((NC, n_pad, D), jnp.float32),
        scratch_types=[
            pltpu.VMEM((CHUNK,), jnp.int32),
            pltpu.VMEM((CHUNK,), jnp.int32),
            pltpu.VMEM((CHUNK,), jnp.float32),
            pltpu.VMEM((CHUNK, D), jnp.float32),
            pltpu.VMEM_SHARED((n_pad, D), jnp.float32),
            pltpu.SemaphoreType.DMA,
        ],
    )
    def k(t_hbm, src_hbm, dst_hbm, attr_hbm, z_hbm, out_hbm,
          src_v, dst_v, attr_v, rows_v, acc_sh, sem):
        cid = lax.axis_index("c")
        sid = lax.axis_index("s")
        wid = cid * NS + sid

        pltpu.sync_copy(z_hbm, acc_sh.at[pl.ds(sid * rows_per_tile,
                                               rows_per_tile)])
        plsc.subcore_barrier()

        def body(i, carry):
            base = (wid * chunks + i) * CHUNK
            pltpu.sync_copy(src_hbm.at[pl.ds(base, CHUNK)], src_v)
            pltpu.sync_copy(dst_hbm.at[pl.ds(base, CHUNK)], dst_v)
            pltpu.sync_copy(attr_hbm.at[pl.ds(base, CHUNK)], attr_v)
            pltpu.async_copy(t_hbm.at[src_v], rows_v, sem).wait()
            for g in range(CHUNK // 16):
                av = attr_v[pl.ds(g * 16, 16)]
                for r in range(16):
                    e = g * 16 + r
                    scale = av.at[jnp.full((16,), r, jnp.int32)].get(
                        mode="promise_in_bounds")
                    for j in range(D // 16):
                        v = rows_v[e, pl.ds(j * 16, 16)]
                        rows_v[e, pl.ds(j * 16, 16)] = v * scale
            pltpu.sync_copy(rows_v, acc_sh.at[dst_v], add=True)
            return carry

        lax.fori_loop(0, chunks, body, 0)
        plsc.subcore_barrier()
        pltpu.sync_copy(
            acc_sh.at[pl.ds(sid * rows_per_tile, rows_per_tile)],
            out_hbm.at[cid, pl.ds(sid * rows_per_tile, rows_per_tile)])

    return k(t, src, dst, attr, zeros_block)


def _final_body(a_ref, x_ref, wu_ref, bu_ref, g_ref, b_ref, o_ref):
    upd = a_ref[0] + a_ref[1] + x_ref[...]
    h = jnp.dot(upd, wu_ref[...],
                preferred_element_type=jnp.float32) + bu_ref[...]
    mu = jnp.mean(h, axis=-1, keepdims=True)
    var = jnp.mean((h - mu) ** 2, axis=-1, keepdims=True)
    hn = (h - mu) * lax.rsqrt(var + 1e-5)
    o_ref[...] = jnp.maximum(hn * g_ref[...] + b_ref[...], 0.0)


def _final(partials, x, Wu, bu, gamma, beta):
    n = x.shape[0]
    blk = 1000
    return pl.pallas_call(
        _final_body,
        grid=(n // blk,),
        in_specs=[
            pl.BlockSpec((NC, blk, D), lambda i: (0, i, 0)),
            pl.BlockSpec((blk, D), lambda i: (i, 0)),
            pl.BlockSpec((D, D), lambda i: (0, 0)),
            pl.BlockSpec((1, D), lambda i: (0, 0)),
            pl.BlockSpec((1, D), lambda i: (0, 0)),
            pl.BlockSpec((1, D), lambda i: (0, 0)),
        ],
        out_specs=pl.BlockSpec((blk, D), lambda i: (i, 0)),
        out_shape=jax.ShapeDtypeStruct((n, D), jnp.float32),
    )(partials, x, Wu, bu.reshape(1, D), gamma.reshape(1, D),
      beta.reshape(1, D))


def kernel(x, edge_index, edge_attr, W1, b1, W2, b2, Wu, bu, gamma, beta):
    n = x.shape[0]
    e = edge_index.shape[1]
    t = _mlp(x, W1, b1, W2, b2)

    ew = NW * CHUNK
    ep = ((e + ew - 1) // ew) * ew
    pad = ep - e
    src = jnp.pad(edge_index[0].astype(jnp.int32), (0, pad))
    dst = jnp.pad(edge_index[1].astype(jnp.int32), (0, pad))
    attr = jnp.pad(edge_attr, (0, pad))
    n_pad = ((n + NS * 8 - 1) // (NS * 8)) * (NS * 8)
    zeros_block = jnp.zeros((n_pad // NS, D), jnp.float32)

    partials = _sc_scatter(t, src, dst, attr, zeros_block, n_pad)
    return _final(partials, x, Wu, bu, gamma, beta)
